# baseline (speedup 1.0000x reference)
import dataclasses
import functools

import jax
import jax.numpy as jnp
from jax import lax
from jax.experimental import pallas as pl
from jax.experimental.pallas import tpu as pltpu
from jax.experimental.pallas import tpu_sc as plsc

USERS = 25000
ITEMS = 25000
N = USERS + ITEMS
E = 800000
D = 64
DH = 32
NL = 3
B = 1024
NS = 16
CH = 128
NCHUNK = E // CH
NPAD = 50048
ROWS_PER_SUB = NPAD // NS
LAST_ROWS = N - (NS - 1) * ROWS_PER_SUB
ZROWS = 136
UB = B // NS


def _splat(val, dtype):
    return jnp.full((16,), val, dtype)


def _sc_propagate(tab0, edata, uid2d):
    f32 = jnp.float32
    i32 = jnp.int32
    mesh = plsc.VectorSubcoreMesh(core_axis_name="c", subcore_axis_name="s")
    out_type = [jax.ShapeDtypeStruct((2 * N, DH), f32) for _ in range(NL)]
    out_type.append(jax.ShapeDtypeStruct((2 * B, DH), f32))

    cp = pltpu.CompilerParams()
    if "needs_layout_passes" in pltpu.CompilerParams.__dataclass_fields__:
        cp = dataclasses.replace(cp, needs_layout_passes=False)
    if "use_tc_tiling_on_sc" in pltpu.CompilerParams.__dataclass_fields__:
        cp = dataclasses.replace(cp, use_tc_tiling_on_sc=False)

    @functools.partial(
        pl.kernel,
        mesh=mesh,
        out_type=out_type,
        compiler_params=cp,
        scratch_types=[
            pltpu.VMEM_SHARED((NPAD, DH), f32),
            pltpu.VMEM((3, CH), i32),
            pltpu.VMEM((CH, DH), f32),
            pltpu.VMEM((ZROWS, DH), f32),
            pltpu.VMEM((1, UB), i32),
            pltpu.VMEM((UB, DH), f32),
            pltpu.VMEM((UB, DH), f32),
            pltpu.SemaphoreType.DMA,
        ],
    )
    def k(tab_hbm, ed_hbm, uid_hbm, o1, o2, o3, uout,
          acc, edv, rows, zbuf, uidv, ubuf, uacc, sem):
        c = lax.axis_index("c")
        s = lax.axis_index("s")
        cbase = c * N

        zv = jnp.zeros((16,), f32)

        @pl.loop(0, ZROWS)
        def _(r):
            for dd in range(0, DH, 16):
                zbuf[r, pl.ds(dd, 16)] = zv

        @pl.loop(0, UB)
        def _(r):
            for dd in range(0, DH, 16):
                uacc[r, pl.ds(dd, 16)] = zv

        pltpu.sync_copy(uid_hbm.at[s], uidv.at[0])
        for t in range(0, UB, 16):
            uidv[0, pl.ds(t, 16)] = uidv[0, pl.ds(t, 16)] + _splat(cbase, i32)

        rbase = s * ROWS_PER_SUB
        tabs = [tab_hbm, o1, o2]
        louts = [o1, o2, o3]
        for li in range(NL):
            tab = tabs[li]
            out = louts[li]

            for z in range(ROWS_PER_SUB // ZROWS):
                pltpu.sync_copy(zbuf, acc.at[pl.ds(rbase + z * ZROWS, ZROWS)])
            plsc.subcore_barrier()

            @pl.loop(0, NCHUNK // NS + 1)
            def _(it):
                j = it * NS + s

                @pl.when(j < NCHUNK)
                def _():
                    pltpu.sync_copy(ed_hbm.at[j], edv)
                    for t in range(0, CH, 16):
                        edv[0, pl.ds(t, 16)] = (
                            edv[0, pl.ds(t, 16)] + _splat(cbase, i32))
                    pltpu.async_copy(tab.at[edv.at[0]], rows, sem).wait()

                    @pl.loop(0, CH, step=8)
                    def _(e0):
                        for k in range(8):
                            e = e0 + k
                            wbits = plsc.load_gather(
                                edv, [_splat(2, i32), _splat(e, i32)])
                            wv = plsc.bitcast(wbits, f32)
                            for dd in range(0, DH, 16):
                                rows[e, pl.ds(dd, 16)] = (
                                    rows[e, pl.ds(dd, 16)] * wv)

                    pltpu.sync_copy(rows, acc.at[edv.at[1]], add=True)

            plsc.subcore_barrier()

            @pl.when(s < NS - 1)
            def _():
                pltpu.sync_copy(acc.at[pl.ds(rbase, ROWS_PER_SUB)],
                                out.at[pl.ds(cbase + rbase, ROWS_PER_SUB)])

            @pl.when(s == NS - 1)
            def _():
                pltpu.sync_copy(acc.at[pl.ds(rbase, LAST_ROWS)],
                                out.at[pl.ds(cbase + rbase, LAST_ROWS)])

            plsc.subcore_barrier()

            pltpu.async_copy(out.at[uidv.at[0]], ubuf, sem).wait()

            @pl.loop(0, UB)
            def _(r):
                for dd in range(0, DH, 16):
                    uacc[r, pl.ds(dd, 16)] = (
                        uacc[r, pl.ds(dd, 16)] + ubuf[r, pl.ds(dd, 16)])

        pltpu.sync_copy(uacc, uout.at[pl.ds(c * B + s * UB, UB)])

    return k(tab0, edata, uid2d)


def _tc_sum_items(o1, o2, o3):
    f32 = jnp.float32
    RB = 5000

    def body(al, bl, cl, ah, bh, ch, out_ref):
        sl = (al[...] + bl[...] + cl[...]) * (1.0 / 3.0)
        sh = (ah[...] + bh[...] + ch[...]) * (1.0 / 3.0)
        out_ref[...] = jnp.concatenate([sl, sh], axis=1)

    lo = pl.BlockSpec((RB, DH), lambda i: (USERS // RB + i, 0))
    hi = pl.BlockSpec((RB, DH), lambda i: ((N + USERS) // RB + i, 0))
    return pl.pallas_call(
        body,
        grid=(ITEMS // RB,),
        in_specs=[lo, lo, lo, hi, hi, hi],
        out_specs=pl.BlockSpec((RB, D), lambda i: (i, 0)),
        out_shape=jax.ShapeDtypeStruct((ITEMS, D), f32),
    )(o1, o2, o3, o1, o2, o3)


def _tc_rating(si, uout):
    f32 = jnp.float32
    RB = 64
    NBLK = B // RB

    def body(s, ul, uh, out_ref):
        u = jnp.concatenate([ul[...], uh[...]], axis=1) * (1.0 / 3.0)
        dn = (((1,), (1,)), ((), ()))
        acc = lax.dot_general(u, s[...], dn, preferred_element_type=f32)
        out_ref[...] = jax.nn.sigmoid(acc)

    sspec = pl.BlockSpec((ITEMS, D), lambda i: (0, 0))
    ulo = pl.BlockSpec((RB, DH), lambda i: (i, 0))
    uhi = pl.BlockSpec((RB, DH), lambda i: (NBLK + i, 0))
    return pl.pallas_call(
        body,
        grid=(NBLK,),
        in_specs=[sspec, ulo, uhi],
        out_specs=pl.BlockSpec((RB, ITEMS), lambda i: (i, 0)),
        out_shape=jax.ShapeDtypeStruct((B, ITEMS), f32),
    )(si, uout, uout)


def kernel(user_emb, item_emb, edge_src, edge_dst, edge_weight, user_ids):
    f32 = jnp.float32
    i32 = jnp.int32
    user_emb = user_emb.astype(f32)
    item_emb = item_emb.astype(f32)
    src = edge_src.astype(i32)
    dst = edge_dst.astype(i32)
    w = edge_weight.astype(f32)
    uid = user_ids.astype(i32)

    tab0 = jnp.concatenate(
        [user_emb[:, :DH], item_emb[:, :DH], user_emb[:, DH:], item_emb[:, DH:]],
        axis=0)
    edata = jnp.stack(
        [src.reshape(NCHUNK, CH), dst.reshape(NCHUNK, CH),
         lax.bitcast_convert_type(w, i32).reshape(NCHUNK, CH)], axis=1)
    uid2d = uid.reshape(NS, UB)

    o1, o2, o3, uout = _sc_propagate(tab0, edata, uid2d)
    return _tc_rating(_tc_sum_items(o1, o2, o3), uout)


# speedup vs baseline: 1.8285x; 1.8285x over previous
import dataclasses
import functools

import jax
import jax.numpy as jnp
from jax import lax
from jax.experimental import pallas as pl
from jax.experimental.pallas import tpu as pltpu
from jax.experimental.pallas import tpu_sc as plsc

USERS = 25000
ITEMS = 25000
N = USERS + ITEMS
E = 800000
D = 64
DH = 32
NL = 3
B = 1024
NS = 16
CH = 128
NCHUNK = E // CH
NITER = 392
NCHP = NITER * NS
NPAD = 50048
ROWS_PER_SUB = NPAD // NS
LAST_ROWS = N - (NS - 1) * ROWS_PER_SUB
ZROWS = 136
UB = B // NS


def _splat(val, dtype):
    return jnp.full((16,), val, dtype)


def _sc_propagate(tab0, edata, uid2d):
    f32 = jnp.float32
    i32 = jnp.int32
    mesh = plsc.VectorSubcoreMesh(core_axis_name="c", subcore_axis_name="s")
    out_type = [jax.ShapeDtypeStruct((2 * N, DH), f32) for _ in range(NL)]
    out_type.append(jax.ShapeDtypeStruct((2 * B, DH), f32))

    cp = pltpu.CompilerParams()
    if "needs_layout_passes" in pltpu.CompilerParams.__dataclass_fields__:
        cp = dataclasses.replace(cp, needs_layout_passes=False)
    if "use_tc_tiling_on_sc" in pltpu.CompilerParams.__dataclass_fields__:
        cp = dataclasses.replace(cp, use_tc_tiling_on_sc=False)

    @functools.partial(
        pl.kernel,
        mesh=mesh,
        out_type=out_type,
        compiler_params=cp,
        scratch_types=[
            pltpu.VMEM_SHARED((NPAD, DH), f32),
            pltpu.VMEM((3, CH), i32),
            pltpu.VMEM((3, CH), i32),
            pltpu.VMEM((3, CH), i32),
            pltpu.VMEM((CH, DH), f32),
            pltpu.VMEM((CH, DH), f32),
            pltpu.VMEM((ZROWS, DH), f32),
            pltpu.VMEM((1, UB), i32),
            pltpu.VMEM((UB, DH), f32),
            pltpu.VMEM((UB, DH), f32),
            pltpu.SemaphoreType.DMA,
            pltpu.SemaphoreType.DMA,
            pltpu.SemaphoreType.DMA,
            pltpu.SemaphoreType.DMA,
            pltpu.SemaphoreType.DMA,
            pltpu.SemaphoreType.DMA,
        ],
    )
    def k(tab_hbm, ed_hbm, uid_hbm, o1, o2, o3, uout,
          acc, edv0, edv1, edv2, rows0, rows1, zbuf, uidv, ubuf, uacc,
          sed0, sed1, sed2, sg0, sg1, sem):
        c = lax.axis_index("c")
        s = lax.axis_index("s")
        cbase = c * N

        zv = jnp.zeros((16,), f32)

        @pl.loop(0, ZROWS)
        def _(r):
            for dd in range(0, DH, 16):
                zbuf[r, pl.ds(dd, 16)] = zv

        @pl.loop(0, UB)
        def _(r):
            for dd in range(0, DH, 16):
                uacc[r, pl.ds(dd, 16)] = zv

        pltpu.sync_copy(uid_hbm.at[s], uidv.at[0])
        for t in range(0, UB, 16):
            uidv[0, pl.ds(t, 16)] = uidv[0, pl.ds(t, 16)] + _splat(cbase, i32)

        rbase = s * ROWS_PER_SUB
        tabs = [tab_hbm, o1, o2]
        louts = [o1, o2, o3]
        for li in range(NL):
            tab = tabs[li]
            out = louts[li]

            for z in range(ROWS_PER_SUB // ZROWS):
                pltpu.sync_copy(zbuf, acc.at[pl.ds(rbase + z * ZROWS, ZROWS)])
            plsc.subcore_barrier()

            eb = (edv0, edv1, edv2)
            sed = (sed0, sed1, sed2)
            rb = (rows0, rows1)
            sg = (sg0, sg1)

            def _records(i, r):
                pltpu.async_copy(ed_hbm.at[i * NS + s], eb[r], sed[r])

            def _wait_records(r):
                pltpu.make_async_copy(ed_hbm.at[0], eb[r], sed[r]).wait()

            def _adjust(r):
                for t in range(0, CH, 16):
                    eb[r][0, pl.ds(t, 16)] = (
                        eb[r][0, pl.ds(t, 16)] + _splat(cbase, i32))

            def _gather(r, q):
                pltpu.async_copy(tab.at[eb[r].at[0]], rb[q], sg[q])

            def _wait_gather(r, q):
                pltpu.make_async_copy(tab.at[eb[r].at[0]], rb[q], sg[q]).wait()

            def _scale_scatter(r, q):
                edv = eb[r]
                rows = rb[q]

                @pl.loop(0, CH, step=8)
                def _(e0):
                    for k in range(8):
                        e = e0 + k
                        wbits = plsc.load_gather(
                            edv, [_splat(2, i32), _splat(e, i32)])
                        wv = plsc.bitcast(wbits, f32)
                        for dd in range(0, DH, 16):
                            rows[e, pl.ds(dd, 16)] = (
                                rows[e, pl.ds(dd, 16)] * wv)

                pltpu.sync_copy(rows, acc.at[edv.at[1]], add=True)

            pltpu.sync_copy(ed_hbm.at[s], edv0)
            _adjust(0)
            _gather(0, 0)
            _records(1, 1)

            @pl.loop(0, NITER - 2, step=6)
            def _(i0):
                for u in range(6):
                    rc = u % 3
                    q = u % 2
                    i = i0 + u
                    _wait_records((rc + 1) % 3)
                    _adjust((rc + 1) % 3)
                    _gather((rc + 1) % 3, q ^ 1)
                    _records(i + 2, (rc + 2) % 3)
                    _wait_gather(rc, q)
                    _scale_scatter(rc, q)

            _wait_records(1)
            _adjust(1)
            _gather(1, 1)
            _wait_gather(0, 0)
            _scale_scatter(0, 0)
            _wait_gather(1, 1)
            _scale_scatter(1, 1)

            plsc.subcore_barrier()

            @pl.when(s < NS - 1)
            def _():
                pltpu.sync_copy(acc.at[pl.ds(rbase, ROWS_PER_SUB)],
                                out.at[pl.ds(cbase + rbase, ROWS_PER_SUB)])

            @pl.when(s == NS - 1)
            def _():
                pltpu.sync_copy(acc.at[pl.ds(rbase, LAST_ROWS)],
                                out.at[pl.ds(cbase + rbase, LAST_ROWS)])

            plsc.subcore_barrier()

            pltpu.async_copy(out.at[uidv.at[0]], ubuf, sem).wait()

            @pl.loop(0, UB)
            def _(r):
                for dd in range(0, DH, 16):
                    uacc[r, pl.ds(dd, 16)] = (
                        uacc[r, pl.ds(dd, 16)] + ubuf[r, pl.ds(dd, 16)])

        pltpu.sync_copy(uacc, uout.at[pl.ds(c * B + s * UB, UB)])

    return k(tab0, edata, uid2d)


def _tc_sum_items(o1, o2, o3):
    f32 = jnp.float32
    RB = 5000

    def body(al, bl, cl, ah, bh, ch, out_ref):
        sl = (al[...] + bl[...] + cl[...]) * (1.0 / 3.0)
        sh = (ah[...] + bh[...] + ch[...]) * (1.0 / 3.0)
        out_ref[...] = jnp.concatenate([sl, sh], axis=1)

    lo = pl.BlockSpec((RB, DH), lambda i: (USERS // RB + i, 0))
    hi = pl.BlockSpec((RB, DH), lambda i: ((N + USERS) // RB + i, 0))
    return pl.pallas_call(
        body,
        grid=(ITEMS // RB,),
        in_specs=[lo, lo, lo, hi, hi, hi],
        out_specs=pl.BlockSpec((RB, D), lambda i: (i, 0)),
        out_shape=jax.ShapeDtypeStruct((ITEMS, D), f32),
    )(o1, o2, o3, o1, o2, o3)


def _tc_rating(si, uout):
    f32 = jnp.float32
    RB = 64
    NBLK = B // RB

    def body(s, ul, uh, out_ref):
        u = jnp.concatenate([ul[...], uh[...]], axis=1) * (1.0 / 3.0)
        dn = (((1,), (1,)), ((), ()))
        acc = lax.dot_general(u, s[...], dn, preferred_element_type=f32)
        out_ref[...] = jax.nn.sigmoid(acc)

    sspec = pl.BlockSpec((ITEMS, D), lambda i: (0, 0))
    ulo = pl.BlockSpec((RB, DH), lambda i: (i, 0))
    uhi = pl.BlockSpec((RB, DH), lambda i: (NBLK + i, 0))
    return pl.pallas_call(
        body,
        grid=(NBLK,),
        in_specs=[sspec, ulo, uhi],
        out_specs=pl.BlockSpec((RB, ITEMS), lambda i: (i, 0)),
        out_shape=jax.ShapeDtypeStruct((B, ITEMS), f32),
    )(si, uout, uout)


def kernel(user_emb, item_emb, edge_src, edge_dst, edge_weight, user_ids):
    f32 = jnp.float32
    i32 = jnp.int32
    user_emb = user_emb.astype(f32)
    item_emb = item_emb.astype(f32)
    src = edge_src.astype(i32)
    dst = edge_dst.astype(i32)
    w = edge_weight.astype(f32)
    uid = user_ids.astype(i32)

    tab0 = jnp.concatenate(
        [user_emb[:, :DH], item_emb[:, :DH], user_emb[:, DH:], item_emb[:, DH:]],
        axis=0)
    edata = jnp.stack(
        [src.reshape(NCHUNK, CH), dst.reshape(NCHUNK, CH),
         lax.bitcast_convert_type(w, i32).reshape(NCHUNK, CH)], axis=1)
    edata = jnp.concatenate(
        [edata, jnp.zeros((NCHP - NCHUNK, 3, CH), i32)], axis=0)
    uid2d = uid.reshape(NS, UB)

    o1, o2, o3, uout = _sc_propagate(tab0, edata, uid2d)
    return _tc_rating(_tc_sum_items(o1, o2, o3), uout)
